# Initial kernel scaffold; baseline (speedup 1.0000x reference)
import jax
import jax.numpy as jnp
from jax.experimental import pallas as pl


def kernel(in_feat, exteraVar1, edge_index, Ws1, Wn1, b1, Ws2, Wn2, b2, Ws3, Wn3, b3, Ws4, Wn4, b4, Ws5, Wn5, b5, Ws6, Wn6, b6):
    raise NotImplementedError("write your pallas kernel here")


# speedup vs baseline: 2.4292x; 2.4292x over previous
import functools

import jax
import jax.numpy as jnp
from jax import lax
from jax.experimental import pallas as pl
from jax.experimental.pallas import tpu as pltpu
from jax.experimental.pallas import tpu_sc as plsc

_N = 10000
_E = 160000
_CH = 128
_EP = 163840
_EPS_F = 10240
_EPS_E = 5120
_NACC = 10112
_BR = 1000
_W = 128


def _agg_body(y_hbm, src_hbm, dst_hbm, zeros_hbm, out_hbm,
              src_v, dst_v, rows_v, acc, sem, *, feature_split):
    c = lax.axis_index("c")
    s = lax.axis_index("s")
    pltpu.sync_copy(zeros_hbm.at[pl.ds(s * 632, 632)],
                    acc.at[pl.ds(s * 632, 632)])
    plsc.subcore_barrier()
    if feature_split:
        ebase = s * _EPS_F
        sbase = c * _EP + ebase
        nch = _EPS_F // _CH
    else:
        ebase = (c * 16 + s) * _EPS_E
        sbase = ebase
        nch = _EPS_E // _CH

    def body(i, carry):
        off = i * _CH
        pltpu.sync_copy(src_hbm.at[pl.ds(sbase + off, _CH)], src_v)
        pltpu.sync_copy(dst_hbm.at[pl.ds(ebase + off, _CH)], dst_v)
        pltpu.async_copy(y_hbm.at[src_v], rows_v, sem).wait()
        pltpu.sync_copy(rows_v, acc.at[dst_v], add=True)
        return carry

    lax.fori_loop(0, nch, body, 0)
    plsc.subcore_barrier()

    @pl.when(s < 15)
    def _():
        pltpu.sync_copy(acc.at[pl.ds(s * 632, 632)],
                        out_hbm.at[pl.ds(c * _N + s * 632, 632)])

    @pl.when(s == 15)
    def _():
        pltpu.sync_copy(acc.at[pl.ds(15 * 632, 520)],
                        out_hbm.at[pl.ds(c * _N + 15 * 632, 520)])


@functools.cache
def _agg(feature_split):
    mesh = plsc.VectorSubcoreMesh(core_axis_name="c", subcore_axis_name="s")
    return functools.partial(
        pl.kernel,
        out_type=jax.ShapeDtypeStruct((2 * _N, _W), jnp.float32),
        mesh=mesh,
        scratch_types=[
            pltpu.VMEM((_CH,), jnp.int32),
            pltpu.VMEM((_CH,), jnp.int32),
            pltpu.VMEM((_CH, _W), jnp.float32),
            pltpu.VMEM_SHARED((_NACC, _W), jnp.float32),
            pltpu.SemaphoreType.DMA,
        ],
    )(functools.partial(_agg_body, feature_split=feature_split))


def _invdeg(deg_ref):
    return 1.0 / jnp.maximum(deg_ref[0][:, :1] + deg_ref[1][:, :1], 1.0)


def _bs(shape, im):
    return pl.BlockSpec(shape, im)


_IM_ROW = lambda i: (i, 0)
_IM_ALL = lambda i: (0, 0)
_IM_P = lambda i: (0, i, 0)


def _call(body, n_out, douts, ins):
    specs = []
    for a in ins:
        if a.ndim == 3:
            specs.append(_bs((2, _BR, a.shape[2]), _IM_P))
        elif a.shape[0] == _N:
            specs.append(_bs((_BR, a.shape[1]), _IM_ROW))
        else:
            specs.append(_bs(a.shape, _IM_ALL))
    outs = pl.pallas_call(
        body,
        grid=(_N // _BR,),
        in_specs=specs,
        out_specs=[_bs((_BR, d), _IM_ROW) for d in douts],
        out_shape=[jax.ShapeDtypeStruct((_N, d), jnp.float32) for d in douts],
    )(*ins)
    return outs if n_out > 1 else outs[0]


def _mm1_body(x_ref, ws_ref, wn_ref, b_ref, ys_ref, yn0_ref, yn1_ref):
    x = x_ref[...]
    ys_ref[...] = x @ ws_ref[...] + b_ref[...]
    yn = x @ wn_ref[...]
    yn0_ref[...] = yn[:, :_W]
    yn1_ref[...] = yn[:, _W:]


def _fused12_body(ys1_ref, p_ref, deg_ref, ws_ref, wn_ref, b_ref,
                  ys2_ref, yn2_ref):
    agg = jnp.concatenate([p_ref[0], p_ref[1]], axis=-1)
    h1 = jnp.maximum(ys1_ref[...] + agg * _invdeg(deg_ref), 0.0)
    ys2_ref[...] = h1 @ ws_ref[...] + b_ref[...]
    yn2_ref[...] = h1 @ wn_ref[...]


def _comb_body(ys_ref, p_ref, deg_ref, o_ref):
    agg = p_ref[0] + p_ref[1]
    o_ref[...] = jnp.maximum(ys_ref[...] + agg * _invdeg(deg_ref), 0.0)


def _pre_body(x_ref, p_ref, deg_ref, ws_ref, wn_ref, b_ref, o_ref):
    mean = (p_ref[0] + p_ref[1]) * _invdeg(deg_ref)
    r = x_ref[...] @ ws_ref[...] + mean @ wn_ref[...] + b_ref[...]
    o_ref[...] = jnp.maximum(r, 0.0)


def _stitch_body(x_ref, p_ref, deg_ref, ws_ref, wn_ref, b_ref, h3_ref, o_ref):
    mean = (p_ref[0] + p_ref[1]) * _invdeg(deg_ref)
    r = x_ref[...] @ ws_ref[...] + mean @ wn_ref[...] + b_ref[...]
    o_ref[...] = jnp.concatenate(
        [jnp.maximum(r, 0.0), h3_ref[:, :64]], axis=-1)


def _final_body(s6_ref, h2_ref, p_ref, deg_ref, wsa_ref, wsb_ref,
                wna_ref, wnb_ref, b_ref, o_ref):
    inv = _invdeg(deg_ref)
    r = (s6_ref[...] @ wsa_ref[...] + h2_ref[...] @ wsb_ref[...]
         + (p_ref[0] * inv) @ wna_ref[...]
         + (p_ref[1] * inv) @ wnb_ref[...] + b_ref[...])
    o_ref[...] = r


def _padw(w, rows=None):
    din, dout = w.shape
    out = w
    if rows is not None and din < rows:
        out = jnp.concatenate([out, jnp.zeros((rows - din, dout), w.dtype)], 0)
    if dout < _W:
        out = jnp.concatenate(
            [out, jnp.zeros((out.shape[0], _W - dout), w.dtype)], 1)
    return out


def kernel(in_feat, exteraVar1, edge_index, Ws1, Wn1, b1, Ws2, Wn2, b2,
           Ws3, Wn3, b3, Ws4, Wn4, b4, Ws5, Wn5, b5, Ws6, Wn6, b6):
    pad = _EP - _E
    srcp = jnp.concatenate([edge_index[0], jnp.zeros((pad,), jnp.int32)])
    dstp = jnp.concatenate(
        [edge_index[1], _N + (jnp.arange(pad, dtype=jnp.int32) % 16)])
    src2 = jnp.concatenate([srcp, srcp + _N])
    zeros = jnp.zeros((_NACC, _W), jnp.float32)
    ones_t = jnp.ones((_N, _W), jnp.float32)

    aggf = _agg(True)
    agge = _agg(False)

    degp = agge(ones_t, srcp, dstp, zeros).reshape(2, _N, _W)[:, :, :8]

    w3s, w3n = _padw(Ws3), _padw(Wn3)
    b3p = jnp.concatenate([b3, jnp.zeros((64,), b3.dtype)]).reshape(1, _W)
    w4s, w4n = _padw(Ws4, rows=_W), _padw(Wn4, rows=_W)
    b4p = jnp.concatenate([b4, jnp.zeros((64,), b4.dtype)]).reshape(1, _W)
    w4sr = jnp.concatenate([Ws4, jnp.zeros((64, 64), Ws4.dtype)], 0)
    w4nr = jnp.concatenate([Wn4, jnp.zeros((64, 64), Wn4.dtype)], 0)
    w6sa, w6sb = Ws6[:_W], Ws6[_W:]
    w6na, w6nb = Wn6[:_W], Wn6[_W:]
    b1r, b2r, b4r = b1.reshape(1, -1), b2.reshape(1, -1), b4.reshape(1, -1)
    b5r, b6r = b5.reshape(1, -1), b6.reshape(1, -1)

    h = in_feat
    out = None
    for _ in range(2):
        ys1, yn0, yn1 = _call(_mm1_body, 3, (256, _W, _W),
                              (h, Ws1, Wn1, b1r))
        p1 = aggf(jnp.concatenate([yn0, yn1], 0), src2, dstp, zeros)
        ys2, yn2 = _call(_fused12_body, 2, (_W, _W),
                         (ys1, p1.reshape(2, _N, _W), degp, Ws2, Wn2, b2r))
        p2 = agge(yn2, srcp, dstp, zeros).reshape(2, _N, _W)
        h2 = _call(_comb_body, 1, (_W,), (ys2, p2, degp))
        p3 = agge(h2, srcp, dstp, zeros).reshape(2, _N, _W)
        h3p = _call(_pre_body, 1, (_W,), (h2, p3, degp, w3s, w3n, b3p))
        p4 = agge(h3p, srcp, dstp, zeros).reshape(2, _N, _W)
        h4p = _call(_pre_body, 1, (_W,), (h3p, p4, degp, w4s, w4n, b4p))
        p5 = agge(h4p, srcp, dstp, zeros).reshape(2, _N, _W)
        h5 = _call(_stitch_body, 1, (_W,),
                   (h4p, p5, degp, w4sr, w4nr, b4r, h3p))
        p6 = agge(h5, srcp, dstp, zeros).reshape(2, _N, _W)
        s6 = _call(_pre_body, 1, (_W,), (h5, p6, degp, Ws5, Wn5, b5r))
        h6st = jnp.concatenate([s6, h2], 0)
        p7 = aggf(h6st, src2, dstp, zeros).reshape(2, _N, _W)
        out = _call(_final_body, 1, (192,),
                    (s6, h2, p7, degp, w6sa, w6sb, w6na, w6nb, b6r))
        h = jnp.concatenate([out, exteraVar1], axis=1)
    return out
